# baseline (speedup 1.0000x reference)
import functools

import jax
import jax.numpy as jnp
from jax import lax
from jax.experimental import pallas as pl
from jax.experimental.pallas import tpu as pltpu
from jax.experimental.pallas import tpu_sc as plsc

N_NODES = 10000
N_PAD = 10240
D_IN = 128
D_HID = 128
D_OUT = 64
N_EDGES = 320000
NC, NS = 2, 16
NW = NC * NS
EDGES_PER_W = 10240
E_PAD = EDGES_PER_W * NW
ROWS_PER_TILE = N_PAD // NS

KG = 80
CHUNKS_G = EDGES_PER_W // KG
KC = 128
CHUNKS_C = EDGES_PER_W // KC
IDX_BLK = 8

CH0_G, CH1_G = 200, 56
CH_C = 160


def _tc1_body(x_ref, w1_ref, asv_ref, adv_ref, h_ref, as_ref, ad_ref):
    h = jnp.dot(x_ref[...], w1_ref[...], preferred_element_type=jnp.float32)
    h_ref[...] = h
    as_ref[...] = jnp.dot(h, asv_ref[...], preferred_element_type=jnp.float32)
    ad_ref[...] = jnp.dot(h, adv_ref[...], preferred_element_type=jnp.float32)


def _tc1(x_pad, W1, att_src, att_dst):
    grid = (N_PAD // 128,)
    return pl.pallas_call(
        _tc1_body,
        grid=grid,
        in_specs=[
            pl.BlockSpec((128, D_IN), lambda i: (i, 0)),
            pl.BlockSpec((D_IN, D_HID), lambda i: (0, 0)),
            pl.BlockSpec((D_HID, 1), lambda i: (0, 0)),
            pl.BlockSpec((D_HID, 1), lambda i: (0, 0)),
        ],
        out_specs=[
            pl.BlockSpec((128, D_HID), lambda i: (i, 0)),
            pl.BlockSpec((128, 1), lambda i: (i, 0)),
            pl.BlockSpec((128, 1), lambda i: (i, 0)),
        ],
        out_shape=[
            jax.ShapeDtypeStruct((N_PAD, D_HID), jnp.float32),
            jax.ShapeDtypeStruct((N_PAD, 1), jnp.float32),
            jax.ShapeDtypeStruct((N_PAD, 1), jnp.float32),
        ],
    )(x_pad, W1, att_src.reshape(D_HID, 1), att_dst.reshape(D_HID, 1))


def _sc_gat_body(src2d, dst2d, asrc_hbm, adst_hbm, h_hbm,
                 out1_hbm, den_hbm, deg_hbm,
                 sidx_v, didx_v, asrc_t, adst_t, ev0, ev1, ones_v,
                 rb0, rb1, acc1, accden, accdeg, sem0, sem1, ss0, ss1):
    c = lax.axis_index("c")
    s = lax.axis_index("s")
    wid = c * NS + s

    def zrow(i, _):
        for f in range(D_HID // 16):
            rb0[i, pl.ds(f * 16, 16)] = jnp.zeros((16,), jnp.float32)
        return 0
    lax.fori_loop(0, KG, zrow, 0)
    for j in range(KG // 16):
        ev0[pl.ds(j * 16, 16)] = jnp.zeros((16,), jnp.float32)
        ones_v[pl.ds(j * 16, 16)] = jnp.ones((16,), jnp.float32)
    for t in range(ROWS_PER_TILE // KG):
        pltpu.sync_copy(rb0, acc1.at[pl.ds(s * ROWS_PER_TILE + t * KG, KG)])
        pltpu.sync_copy(ev0, accden.at[pl.ds(s * ROWS_PER_TILE + t * KG, KG)])
        pltpu.sync_copy(ev0, accdeg.at[pl.ds(s * ROWS_PER_TILE + t * KG, KG)])
    plsc.subcore_barrier()

    pltpu.sync_copy(asrc_hbm, asrc_t)
    pltpu.sync_copy(adst_hbm, adst_t)

    def compute_e(g, ev):
        def ebody(j, _):
            sv = sidx_v[g, pl.ds(j * 16, 16)]
            dv = didx_v[g, pl.ds(j * 16, 16)]
            va = plsc.load_gather(asrc_t, [sv])
            vb = plsc.load_gather(adst_t, [dv])
            al = va + vb
            al = jnp.where(al > 0, al, 0.2 * al)
            ev[pl.ds(j * 16, 16)] = jnp.exp(al)
            return 0
        lax.fori_loop(0, KG // 16, ebody, 0, unroll=KG // 16)

    def scale(rb, ev):
        def sbody(i, _):
            ii = jax.lax.broadcast(i, (16,)).astype(jnp.int32)
            eb = plsc.load_gather(ev, [ii])
            for f in range(D_HID // 16):
                rb[i, pl.ds(f * 16, 16)] = rb[i, pl.ds(f * 16, 16)] * eb
            return 0
        lax.fori_loop(0, KG, sbody, 0, unroll=2)

    chunk0 = jnp.where(c == 0, s * CH0_G, NS * CH0_G + s * CH1_G)
    nblk = jnp.where(c == 0, CH0_G // IDX_BLK, CH1_G // IDX_BLK)

    def blk_body(b, _):
        base = chunk0 + b * IDX_BLK
        pltpu.sync_copy(src2d.at[pl.ds(base, IDX_BLK)], sidx_v)
        pltpu.sync_copy(dst2d.at[pl.ds(base, IDX_BLK)], didx_v)

        pltpu.async_copy(h_hbm.at[sidx_v.at[0]], rb0, sem0)

        def pair_body(p, _):
            g0 = 2 * p
            g1 = 2 * p + 1

            @pl.when(p > 0)
            def _():
                pltpu.make_async_copy(rb1, acc1.at[didx_v.at[g1]], ss1).wait()

            pltpu.async_copy(h_hbm.at[sidx_v.at[g1]], rb1, sem1)

            compute_e(g0, ev0)
            pltpu.sync_copy(ev0, accden.at[didx_v.at[g0]], add=True)
            pltpu.sync_copy(ones_v, accdeg.at[didx_v.at[g0]], add=True)

            pltpu.make_async_copy(h_hbm.at[sidx_v.at[g0]], rb0, sem0).wait()
            scale(rb0, ev0)
            pltpu.async_copy(rb0, acc1.at[didx_v.at[g0]], ss0, add=True)

            compute_e(g1, ev1)
            pltpu.sync_copy(ev1, accden.at[didx_v.at[g1]], add=True)
            pltpu.sync_copy(ones_v, accdeg.at[didx_v.at[g1]], add=True)

            @pl.when(p < IDX_BLK // 2 - 1)
            def _():
                pltpu.make_async_copy(rb0, acc1.at[didx_v.at[g0]], ss0).wait()
                pltpu.async_copy(h_hbm.at[sidx_v.at[g0 + 2]], rb0, sem0)

            pltpu.make_async_copy(h_hbm.at[sidx_v.at[g1]], rb1, sem1).wait()
            scale(rb1, ev1)
            pltpu.async_copy(rb1, acc1.at[didx_v.at[g1]], ss1, add=True)
            return 0

        lax.fori_loop(0, IDX_BLK // 2, pair_body, 0)
        pltpu.make_async_copy(rb0, acc1.at[didx_v.at[0]], ss0).wait()
        pltpu.make_async_copy(rb1, acc1.at[didx_v.at[0]], ss1).wait()
        return 0

    lax.fori_loop(0, nblk, blk_body, 0)
    plsc.subcore_barrier()

    off = c * N_PAD + s * ROWS_PER_TILE
    pltpu.sync_copy(acc1.at[pl.ds(s * ROWS_PER_TILE, ROWS_PER_TILE)],
                    out1_hbm.at[pl.ds(off, ROWS_PER_TILE)])
    pltpu.sync_copy(accden.at[pl.ds(s * ROWS_PER_TILE, ROWS_PER_TILE)],
                    den_hbm.at[pl.ds(off, ROWS_PER_TILE)])
    pltpu.sync_copy(accdeg.at[pl.ds(s * ROWS_PER_TILE, ROWS_PER_TILE)],
                    deg_hbm.at[pl.ds(off, ROWS_PER_TILE)])


def _sc_gat(src2d, dst2d, a_src, a_dst, h):
    mesh = plsc.VectorSubcoreMesh(core_axis_name="c", subcore_axis_name="s",
                                  num_cores=NC, num_subcores=NS)
    f = pl.kernel(
        _sc_gat_body,
        out_type=[
            jax.ShapeDtypeStruct((NC * N_PAD, D_HID), jnp.float32),
            jax.ShapeDtypeStruct((NC * N_PAD,), jnp.float32),
            jax.ShapeDtypeStruct((NC * N_PAD,), jnp.float32),
        ],
        mesh=mesh,
        scratch_types=[
            pltpu.VMEM((IDX_BLK, KG), jnp.int32),
            pltpu.VMEM((IDX_BLK, KG), jnp.int32),
            pltpu.VMEM((N_PAD,), jnp.float32),
            pltpu.VMEM((N_PAD,), jnp.float32),
            pltpu.VMEM((KG,), jnp.float32),
            pltpu.VMEM((KG,), jnp.float32),
            pltpu.VMEM((KG,), jnp.float32),
            pltpu.VMEM((KG, D_HID), jnp.float32),
            pltpu.VMEM((KG, D_HID), jnp.float32),
            pltpu.VMEM_SHARED((N_PAD, D_HID), jnp.float32),
            pltpu.VMEM_SHARED((N_PAD,), jnp.float32),
            pltpu.VMEM_SHARED((N_PAD,), jnp.float32),
            pltpu.SemaphoreType.DMA,
            pltpu.SemaphoreType.DMA,
            pltpu.SemaphoreType.DMA,
            pltpu.SemaphoreType.DMA,
        ],
        compiler_params=pltpu.CompilerParams(needs_layout_passes=False),
    )
    return f(src2d, dst2d, a_src, a_dst, h)


def _tc2_body(p0_ref, p1_ref, h_ref, as_ref, ad_ref, d0_ref, d1_ref,
              g0_ref, g1_ref, b1_ref, w2_ref, h2p_ref, dinv_ref):
    al = as_ref[...] + ad_ref[...]
    al = jnp.where(al > 0, al, 0.2 * al)
    eself = jnp.exp(al)
    num = p0_ref[0] + p1_ref[0] + eself * h_ref[...]
    den = d0_ref[0] + d1_ref[0] + eself + 1e-16
    out1 = num / den + b1_ref[...]
    h1 = jnp.maximum(out1, 0.0)
    h2 = jnp.dot(h1, w2_ref[...], preferred_element_type=jnp.float32)
    deg = g0_ref[0] + g1_ref[0] + 1.0
    dinv = lax.rsqrt(deg)
    h2p_ref[...] = jnp.concatenate(
        [h2 * dinv, jnp.zeros((128, D_HID - D_OUT), jnp.float32)], axis=1)
    dinv_ref[...] = dinv


def _tc2(out1p, h, a_s, a_d, denp, degp, b1, W2):
    grid = (N_PAD // 128,)
    return pl.pallas_call(
        _tc2_body,
        grid=grid,
        in_specs=[
            pl.BlockSpec((1, 128, D_HID), lambda i: (0, i, 0)),
            pl.BlockSpec((1, 128, D_HID), lambda i: (1, i, 0)),
            pl.BlockSpec((128, D_HID), lambda i: (i, 0)),
            pl.BlockSpec((128, 1), lambda i: (i, 0)),
            pl.BlockSpec((128, 1), lambda i: (i, 0)),
            pl.BlockSpec((1, 128, 1), lambda i: (0, i, 0)),
            pl.BlockSpec((1, 128, 1), lambda i: (1, i, 0)),
            pl.BlockSpec((1, 128, 1), lambda i: (0, i, 0)),
            pl.BlockSpec((1, 128, 1), lambda i: (1, i, 0)),
            pl.BlockSpec((1, D_HID), lambda i: (0, 0)),
            pl.BlockSpec((D_HID, D_OUT), lambda i: (0, 0)),
        ],
        out_specs=[
            pl.BlockSpec((128, D_HID), lambda i: (i, 0)),
            pl.BlockSpec((128, 1), lambda i: (i, 0)),
        ],
        out_shape=[
            jax.ShapeDtypeStruct((N_PAD, D_HID), jnp.float32),
            jax.ShapeDtypeStruct((N_PAD, 1), jnp.float32),
        ],
    )(out1p.reshape(NC, N_PAD, D_HID), out1p.reshape(NC, N_PAD, D_HID),
      h, a_s, a_d,
      denp.reshape(NC, N_PAD, 1), denp.reshape(NC, N_PAD, 1),
      degp.reshape(NC, N_PAD, 1), degp.reshape(NC, N_PAD, 1),
      b1.reshape(1, D_HID), W2)


def _sc_gcn_body(src2d, dst2d, h2p_hbm, out2_hbm,
                 sidx_v, didx_v, rb0, rb1, acc2, sem0, sem1, ss0, ss1):
    s = lax.axis_index("s")

    def zrow(i, _):
        for f in range(D_HID // 16):
            rb0[i, pl.ds(f * 16, 16)] = jnp.zeros((16,), jnp.float32)
        return 0
    lax.fori_loop(0, KC, zrow, 0)
    for t in range(ROWS_PER_TILE // KC):
        pltpu.sync_copy(rb0, acc2.at[pl.ds(s * ROWS_PER_TILE + t * KC, KC)])
    plsc.subcore_barrier()

    chunk0 = s * CH_C
    nblk = CH_C // IDX_BLK

    def blk_body(b, _):
        base = chunk0 + b * IDX_BLK
        pltpu.sync_copy(src2d.at[pl.ds(base, IDX_BLK)], sidx_v)
        pltpu.sync_copy(dst2d.at[pl.ds(base, IDX_BLK)], didx_v)

        pltpu.async_copy(h2p_hbm.at[sidx_v.at[0]], rb0, sem0)

        def pair_body(p, _):
            g0 = 2 * p
            g1 = 2 * p + 1

            @pl.when(p > 0)
            def _():
                pltpu.make_async_copy(rb1, acc2.at[didx_v.at[g1]], ss1).wait()

            pltpu.async_copy(h2p_hbm.at[sidx_v.at[g1]], rb1, sem1)
            pltpu.make_async_copy(h2p_hbm.at[sidx_v.at[g0]], rb0, sem0).wait()
            pltpu.async_copy(rb0, acc2.at[didx_v.at[g0]], ss0, add=True)

            @pl.when(p < IDX_BLK // 2 - 1)
            def _():
                pltpu.make_async_copy(rb0, acc2.at[didx_v.at[g0]], ss0).wait()
                pltpu.async_copy(h2p_hbm.at[sidx_v.at[g0 + 2]], rb0, sem0)

            pltpu.make_async_copy(h2p_hbm.at[sidx_v.at[g1]], rb1, sem1).wait()
            pltpu.async_copy(rb1, acc2.at[didx_v.at[g1]], ss1, add=True)
            return 0

        lax.fori_loop(0, IDX_BLK // 2, pair_body, 0)
        pltpu.make_async_copy(rb0, acc2.at[didx_v.at[0]], ss0).wait()
        pltpu.make_async_copy(rb1, acc2.at[didx_v.at[0]], ss1).wait()
        return 0

    lax.fori_loop(0, nblk, blk_body, 0)
    plsc.subcore_barrier()

    off = s * ROWS_PER_TILE
    pltpu.sync_copy(acc2.at[pl.ds(s * ROWS_PER_TILE, ROWS_PER_TILE)],
                    out2_hbm.at[pl.ds(off, ROWS_PER_TILE)])


def _sc_gcn(src2d, dst2d, h2p):
    mesh = plsc.VectorSubcoreMesh(core_axis_name="c", subcore_axis_name="s",
                                  num_cores=1, num_subcores=NS)
    f = pl.kernel(
        _sc_gcn_body,
        out_type=jax.ShapeDtypeStruct((N_PAD, D_HID), jnp.float32),
        mesh=mesh,
        scratch_types=[
            pltpu.VMEM((IDX_BLK, KC), jnp.int32),
            pltpu.VMEM((IDX_BLK, KC), jnp.int32),
            pltpu.VMEM((KC, D_HID), jnp.float32),
            pltpu.VMEM((KC, D_HID), jnp.float32),
            pltpu.VMEM_SHARED((N_PAD, D_HID), jnp.float32),
            pltpu.SemaphoreType.DMA,
            pltpu.SemaphoreType.DMA,
            pltpu.SemaphoreType.DMA,
            pltpu.SemaphoreType.DMA,
        ],
        compiler_params=pltpu.CompilerParams(needs_layout_passes=False),
    )
    return f(src2d, dst2d, h2p)


def _tc3_body(q0_ref, h2p_ref, dinv_ref, b2_ref, out_ref):
    acc = (q0_ref[...] + h2p_ref[...])[:, :D_OUT]
    out_ref[...] = dinv_ref[...] * acc + b2_ref[...]


def _tc3(out2p, h2p, dinv, b2):
    grid = (N_PAD // 128,)
    return pl.pallas_call(
        _tc3_body,
        grid=grid,
        in_specs=[
            pl.BlockSpec((128, D_HID), lambda i: (i, 0)),
            pl.BlockSpec((128, D_HID), lambda i: (i, 0)),
            pl.BlockSpec((128, 1), lambda i: (i, 0)),
            pl.BlockSpec((1, D_OUT), lambda i: (0, 0)),
        ],
        out_specs=pl.BlockSpec((128, D_OUT), lambda i: (i, 0)),
        out_shape=jax.ShapeDtypeStruct((N_PAD, D_OUT), jnp.float32),
    )(out2p, h2p, dinv, b2.reshape(1, D_OUT))


def kernel(x, edge_index, W1, att_src, att_dst, b1, W2, b2):
    x_pad = jnp.pad(x, ((0, N_PAD - N_NODES), (0, 0)))
    src = edge_index[0].astype(jnp.int32)
    dst = edge_index[1].astype(jnp.int32)
    pad = jnp.full((E_PAD - N_EDGES,), N_PAD - 1, jnp.int32)
    src_flat = jnp.concatenate([src, pad])
    dst_flat = jnp.concatenate([dst, pad])
    src_g = src_flat.reshape(NW * CHUNKS_G, KG)
    dst_g = dst_flat.reshape(NW * CHUNKS_G, KG)
    src_c = src_flat.reshape(NW * CHUNKS_C, KC)
    dst_c = dst_flat.reshape(NW * CHUNKS_C, KC)

    h, a_s, a_d = _tc1(x_pad, W1, att_src, att_dst)
    out1p, denp, degp = _sc_gat(src_g, dst_g,
                                a_s.reshape(N_PAD), a_d.reshape(N_PAD), h)
    h2p, dinv = _tc2(out1p, h, a_s, a_d, denp, degp, b1, W2)
    out2p = _sc_gcn(src_c, dst_c, h2p)
    out2 = _tc3(out2p, h2p, dinv, b2)
    return out2[:N_NODES]


# speedup vs baseline: 1.1094x; 1.1094x over previous
import functools

import jax
import jax.numpy as jnp
from jax import lax
from jax.experimental import pallas as pl
from jax.experimental.pallas import tpu as pltpu
from jax.experimental.pallas import tpu_sc as plsc

N_NODES = 10000
N_PAD = 10240
D_IN = 128
D_HID = 128
D_OUT = 64
N_EDGES = 320000
NC, NS = 2, 16
NW = NC * NS
EDGES_PER_W = 10240
E_PAD = EDGES_PER_W * NW
ROWS_PER_TILE = N_PAD // NS

KG = 80
CHUNKS_G = EDGES_PER_W // KG
KC = 128
CHUNKS_C = EDGES_PER_W // KC
IDX_BLK = 8

CH0_G, CH1_G = 200, 56
CH0_C, CH1_C = 128, 32


def _tc1_body(x_ref, w1_ref, asv_ref, adv_ref, h_ref, as_ref, ad_ref):
    h = jnp.dot(x_ref[...], w1_ref[...], preferred_element_type=jnp.float32)
    h_ref[...] = h
    as_ref[...] = jnp.dot(h, asv_ref[...], preferred_element_type=jnp.float32)
    ad_ref[...] = jnp.dot(h, adv_ref[...], preferred_element_type=jnp.float32)


def _tc1(x_pad, W1, att_src, att_dst):
    grid = (N_PAD // 128,)
    return pl.pallas_call(
        _tc1_body,
        grid=grid,
        in_specs=[
            pl.BlockSpec((128, D_IN), lambda i: (i, 0)),
            pl.BlockSpec((D_IN, D_HID), lambda i: (0, 0)),
            pl.BlockSpec((D_HID, 1), lambda i: (0, 0)),
            pl.BlockSpec((D_HID, 1), lambda i: (0, 0)),
        ],
        out_specs=[
            pl.BlockSpec((128, D_HID), lambda i: (i, 0)),
            pl.BlockSpec((128, 1), lambda i: (i, 0)),
            pl.BlockSpec((128, 1), lambda i: (i, 0)),
        ],
        out_shape=[
            jax.ShapeDtypeStruct((N_PAD, D_HID), jnp.float32),
            jax.ShapeDtypeStruct((N_PAD, 1), jnp.float32),
            jax.ShapeDtypeStruct((N_PAD, 1), jnp.float32),
        ],
    )(x_pad, W1, att_src.reshape(D_HID, 1), att_dst.reshape(D_HID, 1))


def _sc_gat_body(src2d, dst2d, asrc_hbm, adst_hbm, h_hbm,
                 out1_hbm, den_hbm, deg_hbm,
                 sidx_v, didx_v, asrc_t, adst_t, ev0, ev1, ones_v,
                 rb0, rb1, acc1, accden, accdeg, sem0, sem1, ss0, ss1):
    c = lax.axis_index("c")
    s = lax.axis_index("s")
    wid = c * NS + s

    def zrow(i, _):
        for f in range(D_HID // 16):
            rb0[i, pl.ds(f * 16, 16)] = jnp.zeros((16,), jnp.float32)
        return 0
    lax.fori_loop(0, KG, zrow, 0)
    for j in range(KG // 16):
        ev0[pl.ds(j * 16, 16)] = jnp.zeros((16,), jnp.float32)
        ones_v[pl.ds(j * 16, 16)] = jnp.ones((16,), jnp.float32)
    for t in range(ROWS_PER_TILE // KG):
        pltpu.sync_copy(rb0, acc1.at[pl.ds(s * ROWS_PER_TILE + t * KG, KG)])
        pltpu.sync_copy(ev0, accden.at[pl.ds(s * ROWS_PER_TILE + t * KG, KG)])
        pltpu.sync_copy(ev0, accdeg.at[pl.ds(s * ROWS_PER_TILE + t * KG, KG)])
    plsc.subcore_barrier()

    pltpu.sync_copy(asrc_hbm, asrc_t)
    pltpu.sync_copy(adst_hbm, adst_t)

    def compute_e(g, ev):
        def ebody(j, _):
            sv = sidx_v[g, pl.ds(j * 16, 16)]
            dv = didx_v[g, pl.ds(j * 16, 16)]
            va = plsc.load_gather(asrc_t, [sv])
            vb = plsc.load_gather(adst_t, [dv])
            al = va + vb
            al = jnp.where(al > 0, al, 0.2 * al)
            ev[pl.ds(j * 16, 16)] = jnp.exp(al)
            return 0
        lax.fori_loop(0, KG // 16, ebody, 0, unroll=KG // 16)

    def scale(rb, ev):
        def sbody(i, _):
            ii = jax.lax.broadcast(i, (16,)).astype(jnp.int32)
            eb = plsc.load_gather(ev, [ii])
            for f in range(D_HID // 16):
                rb[i, pl.ds(f * 16, 16)] = rb[i, pl.ds(f * 16, 16)] * eb
            return 0
        lax.fori_loop(0, KG, sbody, 0, unroll=2)

    chunk0 = jnp.where(c == 0, s * CH0_G, NS * CH0_G + s * CH1_G)
    nblk = jnp.where(c == 0, CH0_G // IDX_BLK, CH1_G // IDX_BLK)

    def blk_body(b, _):
        base = chunk0 + b * IDX_BLK
        pltpu.sync_copy(src2d.at[pl.ds(base, IDX_BLK)], sidx_v)
        pltpu.sync_copy(dst2d.at[pl.ds(base, IDX_BLK)], didx_v)

        pltpu.async_copy(h_hbm.at[sidx_v.at[0]], rb0, sem0)

        def pair_body(p, _):
            g0 = 2 * p
            g1 = 2 * p + 1

            @pl.when(p > 0)
            def _():
                pltpu.make_async_copy(rb1, acc1.at[didx_v.at[g1]], ss1).wait()

            pltpu.async_copy(h_hbm.at[sidx_v.at[g1]], rb1, sem1)

            compute_e(g0, ev0)
            pltpu.sync_copy(ev0, accden.at[didx_v.at[g0]], add=True)
            pltpu.sync_copy(ones_v, accdeg.at[didx_v.at[g0]], add=True)

            pltpu.make_async_copy(h_hbm.at[sidx_v.at[g0]], rb0, sem0).wait()
            scale(rb0, ev0)
            pltpu.async_copy(rb0, acc1.at[didx_v.at[g0]], ss0, add=True)

            compute_e(g1, ev1)
            pltpu.sync_copy(ev1, accden.at[didx_v.at[g1]], add=True)
            pltpu.sync_copy(ones_v, accdeg.at[didx_v.at[g1]], add=True)

            @pl.when(p < IDX_BLK // 2 - 1)
            def _():
                pltpu.make_async_copy(rb0, acc1.at[didx_v.at[g0]], ss0).wait()
                pltpu.async_copy(h_hbm.at[sidx_v.at[g0 + 2]], rb0, sem0)

            pltpu.make_async_copy(h_hbm.at[sidx_v.at[g1]], rb1, sem1).wait()
            scale(rb1, ev1)
            pltpu.async_copy(rb1, acc1.at[didx_v.at[g1]], ss1, add=True)
            return 0

        lax.fori_loop(0, IDX_BLK // 2, pair_body, 0)
        pltpu.make_async_copy(rb0, acc1.at[didx_v.at[0]], ss0).wait()
        pltpu.make_async_copy(rb1, acc1.at[didx_v.at[0]], ss1).wait()
        return 0

    lax.fori_loop(0, nblk, blk_body, 0)
    plsc.subcore_barrier()

    off = c * N_PAD + s * ROWS_PER_TILE
    pltpu.sync_copy(acc1.at[pl.ds(s * ROWS_PER_TILE, ROWS_PER_TILE)],
                    out1_hbm.at[pl.ds(off, ROWS_PER_TILE)])
    pltpu.sync_copy(accden.at[pl.ds(s * ROWS_PER_TILE, ROWS_PER_TILE)],
                    den_hbm.at[pl.ds(off, ROWS_PER_TILE)])
    pltpu.sync_copy(accdeg.at[pl.ds(s * ROWS_PER_TILE, ROWS_PER_TILE)],
                    deg_hbm.at[pl.ds(off, ROWS_PER_TILE)])


def _sc_gat(src2d, dst2d, a_src, a_dst, h):
    mesh = plsc.VectorSubcoreMesh(core_axis_name="c", subcore_axis_name="s",
                                  num_cores=NC, num_subcores=NS)
    f = pl.kernel(
        _sc_gat_body,
        out_type=[
            jax.ShapeDtypeStruct((NC * N_PAD, D_HID), jnp.float32),
            jax.ShapeDtypeStruct((NC * N_PAD,), jnp.float32),
            jax.ShapeDtypeStruct((NC * N_PAD,), jnp.float32),
        ],
        mesh=mesh,
        scratch_types=[
            pltpu.VMEM((IDX_BLK, KG), jnp.int32),
            pltpu.VMEM((IDX_BLK, KG), jnp.int32),
            pltpu.VMEM((N_PAD,), jnp.float32),
            pltpu.VMEM((N_PAD,), jnp.float32),
            pltpu.VMEM((KG,), jnp.float32),
            pltpu.VMEM((KG,), jnp.float32),
            pltpu.VMEM((KG,), jnp.float32),
            pltpu.VMEM((KG, D_HID), jnp.float32),
            pltpu.VMEM((KG, D_HID), jnp.float32),
            pltpu.VMEM_SHARED((N_PAD, D_HID), jnp.float32),
            pltpu.VMEM_SHARED((N_PAD,), jnp.float32),
            pltpu.VMEM_SHARED((N_PAD,), jnp.float32),
            pltpu.SemaphoreType.DMA,
            pltpu.SemaphoreType.DMA,
            pltpu.SemaphoreType.DMA,
            pltpu.SemaphoreType.DMA,
        ],
        compiler_params=pltpu.CompilerParams(needs_layout_passes=False),
    )
    return f(src2d, dst2d, a_src, a_dst, h)


def _tc2_body(p0_ref, p1_ref, h_ref, as_ref, ad_ref, d0_ref, d1_ref,
              g0_ref, g1_ref, b1_ref, w2_ref, h2p_ref, dinv_ref):
    al = as_ref[...] + ad_ref[...]
    al = jnp.where(al > 0, al, 0.2 * al)
    eself = jnp.exp(al)
    num = p0_ref[0] + p1_ref[0] + eself * h_ref[...]
    den = d0_ref[0] + d1_ref[0] + eself + 1e-16
    out1 = num / den + b1_ref[...]
    h1 = jnp.maximum(out1, 0.0)
    h2 = jnp.dot(h1, w2_ref[...], preferred_element_type=jnp.float32)
    deg = g0_ref[0] + g1_ref[0] + 1.0
    dinv = lax.rsqrt(deg)
    h2p_ref[...] = jnp.concatenate(
        [h2 * dinv, jnp.zeros((128, D_HID - D_OUT), jnp.float32)], axis=1)
    dinv_ref[...] = dinv


def _tc2(out1p, h, a_s, a_d, denp, degp, b1, W2):
    grid = (N_PAD // 128,)
    return pl.pallas_call(
        _tc2_body,
        grid=grid,
        in_specs=[
            pl.BlockSpec((1, 128, D_HID), lambda i: (0, i, 0)),
            pl.BlockSpec((1, 128, D_HID), lambda i: (1, i, 0)),
            pl.BlockSpec((128, D_HID), lambda i: (i, 0)),
            pl.BlockSpec((128, 1), lambda i: (i, 0)),
            pl.BlockSpec((128, 1), lambda i: (i, 0)),
            pl.BlockSpec((1, 128, 1), lambda i: (0, i, 0)),
            pl.BlockSpec((1, 128, 1), lambda i: (1, i, 0)),
            pl.BlockSpec((1, 128, 1), lambda i: (0, i, 0)),
            pl.BlockSpec((1, 128, 1), lambda i: (1, i, 0)),
            pl.BlockSpec((1, D_HID), lambda i: (0, 0)),
            pl.BlockSpec((D_HID, D_OUT), lambda i: (0, 0)),
        ],
        out_specs=[
            pl.BlockSpec((128, D_HID), lambda i: (i, 0)),
            pl.BlockSpec((128, 1), lambda i: (i, 0)),
        ],
        out_shape=[
            jax.ShapeDtypeStruct((N_PAD, D_HID), jnp.float32),
            jax.ShapeDtypeStruct((N_PAD, 1), jnp.float32),
        ],
    )(out1p.reshape(NC, N_PAD, D_HID), out1p.reshape(NC, N_PAD, D_HID),
      h, a_s, a_d,
      denp.reshape(NC, N_PAD, 1), denp.reshape(NC, N_PAD, 1),
      degp.reshape(NC, N_PAD, 1), degp.reshape(NC, N_PAD, 1),
      b1.reshape(1, D_HID), W2)


def _sc_gcn_body(src2d, dst2d, h2p_hbm, out2_hbm,
                 sidx_v, didx_v, rb0, rb1, acc2, sem0, sem1, ss0, ss1):
    c = lax.axis_index("c")
    s = lax.axis_index("s")

    def zrow(i, _):
        for f in range(D_HID // 16):
            rb0[i, pl.ds(f * 16, 16)] = jnp.zeros((16,), jnp.float32)
        return 0
    lax.fori_loop(0, KC, zrow, 0)
    for t in range(ROWS_PER_TILE // KC):
        pltpu.sync_copy(rb0, acc2.at[pl.ds(s * ROWS_PER_TILE + t * KC, KC)])
    plsc.subcore_barrier()

    chunk0 = jnp.where(c == 0, s * CH0_C, NS * CH0_C + s * CH1_C)
    nblk = jnp.where(c == 0, CH0_C // IDX_BLK, CH1_C // IDX_BLK)

    def blk_body(b, _):
        base = chunk0 + b * IDX_BLK
        pltpu.sync_copy(src2d.at[pl.ds(base, IDX_BLK)], sidx_v)
        pltpu.sync_copy(dst2d.at[pl.ds(base, IDX_BLK)], didx_v)

        pltpu.async_copy(h2p_hbm.at[sidx_v.at[0]], rb0, sem0)

        def pair_body(p, _):
            g0 = 2 * p
            g1 = 2 * p + 1

            @pl.when(p > 0)
            def _():
                pltpu.make_async_copy(rb1, acc2.at[didx_v.at[g1]], ss1).wait()

            pltpu.async_copy(h2p_hbm.at[sidx_v.at[g1]], rb1, sem1)
            pltpu.make_async_copy(h2p_hbm.at[sidx_v.at[g0]], rb0, sem0).wait()
            pltpu.async_copy(rb0, acc2.at[didx_v.at[g0]], ss0, add=True)

            @pl.when(p < IDX_BLK // 2 - 1)
            def _():
                pltpu.make_async_copy(rb0, acc2.at[didx_v.at[g0]], ss0).wait()
                pltpu.async_copy(h2p_hbm.at[sidx_v.at[g0 + 2]], rb0, sem0)

            pltpu.make_async_copy(h2p_hbm.at[sidx_v.at[g1]], rb1, sem1).wait()
            pltpu.async_copy(rb1, acc2.at[didx_v.at[g1]], ss1, add=True)
            return 0

        lax.fori_loop(0, IDX_BLK // 2, pair_body, 0)
        pltpu.make_async_copy(rb0, acc2.at[didx_v.at[0]], ss0).wait()
        pltpu.make_async_copy(rb1, acc2.at[didx_v.at[0]], ss1).wait()
        return 0

    lax.fori_loop(0, nblk, blk_body, 0)
    plsc.subcore_barrier()

    off = c * N_PAD + s * ROWS_PER_TILE
    pltpu.sync_copy(acc2.at[pl.ds(s * ROWS_PER_TILE, ROWS_PER_TILE)],
                    out2_hbm.at[pl.ds(off, ROWS_PER_TILE)])


def _sc_gcn(src2d, dst2d, h2p):
    mesh = plsc.VectorSubcoreMesh(core_axis_name="c", subcore_axis_name="s",
                                  num_cores=NC, num_subcores=NS)
    f = pl.kernel(
        _sc_gcn_body,
        out_type=jax.ShapeDtypeStruct((NC * N_PAD, D_HID), jnp.float32),
        mesh=mesh,
        scratch_types=[
            pltpu.VMEM((IDX_BLK, KC), jnp.int32),
            pltpu.VMEM((IDX_BLK, KC), jnp.int32),
            pltpu.VMEM((KC, D_HID), jnp.float32),
            pltpu.VMEM((KC, D_HID), jnp.float32),
            pltpu.VMEM_SHARED((N_PAD, D_HID), jnp.float32),
            pltpu.SemaphoreType.DMA,
            pltpu.SemaphoreType.DMA,
            pltpu.SemaphoreType.DMA,
            pltpu.SemaphoreType.DMA,
        ],
        compiler_params=pltpu.CompilerParams(needs_layout_passes=False),
    )
    return f(src2d, dst2d, h2p)


def _tc3_body(q0_ref, q1_ref, h2p_ref, dinv_ref, b2_ref, out_ref):
    acc = (q0_ref[0] + q1_ref[0] + h2p_ref[...])[:, :D_OUT]
    out_ref[...] = dinv_ref[...] * acc + b2_ref[...]


def _tc3(out2p, h2p, dinv, b2):
    grid = (N_PAD // 128,)
    return pl.pallas_call(
        _tc3_body,
        grid=grid,
        in_specs=[
            pl.BlockSpec((1, 128, D_HID), lambda i: (0, i, 0)),
            pl.BlockSpec((1, 128, D_HID), lambda i: (1, i, 0)),
            pl.BlockSpec((128, D_HID), lambda i: (i, 0)),
            pl.BlockSpec((128, 1), lambda i: (i, 0)),
            pl.BlockSpec((1, D_OUT), lambda i: (0, 0)),
        ],
        out_specs=pl.BlockSpec((128, D_OUT), lambda i: (i, 0)),
        out_shape=jax.ShapeDtypeStruct((N_PAD, D_OUT), jnp.float32),
    )(out2p.reshape(NC, N_PAD, D_HID), out2p.reshape(NC, N_PAD, D_HID),
      h2p, dinv, b2.reshape(1, D_OUT))


def kernel(x, edge_index, W1, att_src, att_dst, b1, W2, b2):
    x_pad = jnp.pad(x, ((0, N_PAD - N_NODES), (0, 0)))
    src = edge_index[0].astype(jnp.int32)
    dst = edge_index[1].astype(jnp.int32)
    pad = jnp.full((E_PAD - N_EDGES,), N_PAD - 1, jnp.int32)
    src_flat = jnp.concatenate([src, pad])
    dst_flat = jnp.concatenate([dst, pad])
    src_g = src_flat.reshape(NW * CHUNKS_G, KG)
    dst_g = dst_flat.reshape(NW * CHUNKS_G, KG)
    src_c = src_flat.reshape(NW * CHUNKS_C, KC)
    dst_c = dst_flat.reshape(NW * CHUNKS_C, KC)

    h, a_s, a_d = _tc1(x_pad, W1, att_src, att_dst)
    out1p, denp, degp = _sc_gat(src_g, dst_g,
                                a_s.reshape(N_PAD), a_d.reshape(N_PAD), h)
    h2p, dinv = _tc2(out1p, h, a_s, a_d, denp, degp, b1, W2)
    out2p = _sc_gcn(src_c, dst_c, h2p)
    out2 = _tc3(out2p, h2p, dinv, b2)
    return out2[:N_NODES]


# speedup vs baseline: 1.1439x; 1.0311x over previous
import functools

import jax
import jax.numpy as jnp
from jax import lax
from jax.experimental import pallas as pl
from jax.experimental.pallas import tpu as pltpu
from jax.experimental.pallas import tpu_sc as plsc

N_NODES = 10000
N_PAD = 10240
D_IN = 128
D_HID = 128
D_OUT = 64
N_EDGES = 320000
NC, NS = 2, 16
NW = NC * NS
EDGES_PER_W = 10240
E_PAD = EDGES_PER_W * NW
ROWS_PER_TILE = N_PAD // NS

KG = 80
CHUNKS_G = EDGES_PER_W // KG
KC = 128
CHUNKS_C = EDGES_PER_W // KC
IDX_BLK = 8

CH0_G, CH1_G = 208, 48
CH0_C, CH1_C = 136, 24


def _tc1_body(x_ref, w1_ref, asv_ref, adv_ref, h_ref, as_ref, ad_ref):
    h = jnp.dot(x_ref[...], w1_ref[...], preferred_element_type=jnp.float32)
    h_ref[...] = h
    as_ref[...] = jnp.dot(h, asv_ref[...], preferred_element_type=jnp.float32)
    ad_ref[...] = jnp.dot(h, adv_ref[...], preferred_element_type=jnp.float32)


def _tc1(x_pad, W1, att_src, att_dst):
    grid = (N_PAD // 128,)
    return pl.pallas_call(
        _tc1_body,
        grid=grid,
        in_specs=[
            pl.BlockSpec((128, D_IN), lambda i: (i, 0)),
            pl.BlockSpec((D_IN, D_HID), lambda i: (0, 0)),
            pl.BlockSpec((D_HID, 1), lambda i: (0, 0)),
            pl.BlockSpec((D_HID, 1), lambda i: (0, 0)),
        ],
        out_specs=[
            pl.BlockSpec((128, D_HID), lambda i: (i, 0)),
            pl.BlockSpec((128, 1), lambda i: (i, 0)),
            pl.BlockSpec((128, 1), lambda i: (i, 0)),
        ],
        out_shape=[
            jax.ShapeDtypeStruct((N_PAD, D_HID), jnp.float32),
            jax.ShapeDtypeStruct((N_PAD, 1), jnp.float32),
            jax.ShapeDtypeStruct((N_PAD, 1), jnp.float32),
        ],
    )(x_pad, W1, att_src.reshape(D_HID, 1), att_dst.reshape(D_HID, 1))


def _sc_gat_body(src2d, dst2d, asrc_hbm, adst_hbm, h_hbm,
                 out1_hbm, den_hbm, deg_hbm,
                 sidx_v, didx_v, asrc_t, adst_t, ev0, ev1, ones_v,
                 rb0, rb1, acc1, accden, accdeg, sem0, sem1, ss0, ss1):
    c = lax.axis_index("c")
    s = lax.axis_index("s")
    wid = c * NS + s

    def zrow(i, _):
        for f in range(D_HID // 16):
            rb0[i, pl.ds(f * 16, 16)] = jnp.zeros((16,), jnp.float32)
        return 0
    lax.fori_loop(0, KG, zrow, 0)
    for j in range(KG // 16):
        ev0[pl.ds(j * 16, 16)] = jnp.zeros((16,), jnp.float32)
        ones_v[pl.ds(j * 16, 16)] = jnp.ones((16,), jnp.float32)
    for t in range(ROWS_PER_TILE // KG):
        pltpu.sync_copy(rb0, acc1.at[pl.ds(s * ROWS_PER_TILE + t * KG, KG)])
        pltpu.sync_copy(ev0, accden.at[pl.ds(s * ROWS_PER_TILE + t * KG, KG)])
        pltpu.sync_copy(ev0, accdeg.at[pl.ds(s * ROWS_PER_TILE + t * KG, KG)])
    plsc.subcore_barrier()

    pltpu.sync_copy(asrc_hbm, asrc_t)
    pltpu.sync_copy(adst_hbm, adst_t)

    def compute_e(g, ev):
        def ebody(j, _):
            sv = sidx_v[g, pl.ds(j * 16, 16)]
            dv = didx_v[g, pl.ds(j * 16, 16)]
            va = plsc.load_gather(asrc_t, [sv])
            vb = plsc.load_gather(adst_t, [dv])
            al = va + vb
            al = jnp.where(al > 0, al, 0.2 * al)
            ev[pl.ds(j * 16, 16)] = jnp.exp(al)
            return 0
        lax.fori_loop(0, KG // 16, ebody, 0, unroll=KG // 16)

    def scale(rb, ev):
        def sbody(i, _):
            ii = jax.lax.broadcast(i, (16,)).astype(jnp.int32)
            eb = plsc.load_gather(ev, [ii])
            for f in range(D_HID // 16):
                rb[i, pl.ds(f * 16, 16)] = rb[i, pl.ds(f * 16, 16)] * eb
            return 0
        lax.fori_loop(0, KG, sbody, 0, unroll=2)

    chunk0 = jnp.where(c == 0, s * CH0_G, NS * CH0_G + s * CH1_G)
    nblk = jnp.where(c == 0, CH0_G // IDX_BLK, CH1_G // IDX_BLK)

    def blk_body(b, _):
        base = chunk0 + b * IDX_BLK
        pltpu.sync_copy(src2d.at[pl.ds(base, IDX_BLK)], sidx_v)
        pltpu.sync_copy(dst2d.at[pl.ds(base, IDX_BLK)], didx_v)

        pltpu.async_copy(h_hbm.at[sidx_v.at[0]], rb0, sem0)

        def pair_body(p, _):
            g0 = 2 * p
            g1 = 2 * p + 1

            @pl.when(p > 0)
            def _():
                pltpu.make_async_copy(rb1, acc1.at[didx_v.at[g1]], ss1).wait()

            pltpu.async_copy(h_hbm.at[sidx_v.at[g1]], rb1, sem1)

            compute_e(g0, ev0)
            pltpu.sync_copy(ev0, accden.at[didx_v.at[g0]], add=True)
            pltpu.sync_copy(ones_v, accdeg.at[didx_v.at[g0]], add=True)

            pltpu.make_async_copy(h_hbm.at[sidx_v.at[g0]], rb0, sem0).wait()
            scale(rb0, ev0)
            pltpu.async_copy(rb0, acc1.at[didx_v.at[g0]], ss0, add=True)

            compute_e(g1, ev1)
            pltpu.sync_copy(ev1, accden.at[didx_v.at[g1]], add=True)
            pltpu.sync_copy(ones_v, accdeg.at[didx_v.at[g1]], add=True)

            @pl.when(p < IDX_BLK // 2 - 1)
            def _():
                pltpu.make_async_copy(rb0, acc1.at[didx_v.at[g0]], ss0).wait()
                pltpu.async_copy(h_hbm.at[sidx_v.at[g0 + 2]], rb0, sem0)

            pltpu.make_async_copy(h_hbm.at[sidx_v.at[g1]], rb1, sem1).wait()
            scale(rb1, ev1)
            pltpu.async_copy(rb1, acc1.at[didx_v.at[g1]], ss1, add=True)
            return 0

        lax.fori_loop(0, IDX_BLK // 2, pair_body, 0)
        pltpu.make_async_copy(rb0, acc1.at[didx_v.at[0]], ss0).wait()
        pltpu.make_async_copy(rb1, acc1.at[didx_v.at[0]], ss1).wait()
        return 0

    lax.fori_loop(0, nblk, blk_body, 0)
    plsc.subcore_barrier()

    off = c * N_PAD + s * ROWS_PER_TILE
    pltpu.sync_copy(acc1.at[pl.ds(s * ROWS_PER_TILE, ROWS_PER_TILE)],
                    out1_hbm.at[pl.ds(off, ROWS_PER_TILE)])
    pltpu.sync_copy(accden.at[pl.ds(s * ROWS_PER_TILE, ROWS_PER_TILE)],
                    den_hbm.at[pl.ds(off, ROWS_PER_TILE)])
    pltpu.sync_copy(accdeg.at[pl.ds(s * ROWS_PER_TILE, ROWS_PER_TILE)],
                    deg_hbm.at[pl.ds(off, ROWS_PER_TILE)])


def _sc_gat(src2d, dst2d, a_src, a_dst, h):
    mesh = plsc.VectorSubcoreMesh(core_axis_name="c", subcore_axis_name="s",
                                  num_cores=NC, num_subcores=NS)
    f = pl.kernel(
        _sc_gat_body,
        out_type=[
            jax.ShapeDtypeStruct((NC * N_PAD, D_HID), jnp.float32),
            jax.ShapeDtypeStruct((NC * N_PAD,), jnp.float32),
            jax.ShapeDtypeStruct((NC * N_PAD,), jnp.float32),
        ],
        mesh=mesh,
        scratch_types=[
            pltpu.VMEM((IDX_BLK, KG), jnp.int32),
            pltpu.VMEM((IDX_BLK, KG), jnp.int32),
            pltpu.VMEM((N_PAD,), jnp.float32),
            pltpu.VMEM((N_PAD,), jnp.float32),
            pltpu.VMEM((KG,), jnp.float32),
            pltpu.VMEM((KG,), jnp.float32),
            pltpu.VMEM((KG,), jnp.float32),
            pltpu.VMEM((KG, D_HID), jnp.float32),
            pltpu.VMEM((KG, D_HID), jnp.float32),
            pltpu.VMEM_SHARED((N_PAD, D_HID), jnp.float32),
            pltpu.VMEM_SHARED((N_PAD,), jnp.float32),
            pltpu.VMEM_SHARED((N_PAD,), jnp.float32),
            pltpu.SemaphoreType.DMA,
            pltpu.SemaphoreType.DMA,
            pltpu.SemaphoreType.DMA,
            pltpu.SemaphoreType.DMA,
        ],
        compiler_params=pltpu.CompilerParams(needs_layout_passes=False),
    )
    return f(src2d, dst2d, a_src, a_dst, h)


def _tc2_body(p0_ref, p1_ref, h_ref, as_ref, ad_ref, d0_ref, d1_ref,
              g0_ref, g1_ref, b1_ref, w2_ref, h2p_ref, dinv_ref):
    al = as_ref[...] + ad_ref[...]
    al = jnp.where(al > 0, al, 0.2 * al)
    eself = jnp.exp(al)
    num = p0_ref[0] + p1_ref[0] + eself * h_ref[...]
    den = d0_ref[0] + d1_ref[0] + eself + 1e-16
    out1 = num / den + b1_ref[...]
    h1 = jnp.maximum(out1, 0.0)
    h2 = jnp.dot(h1, w2_ref[...], preferred_element_type=jnp.float32)
    deg = g0_ref[0] + g1_ref[0] + 1.0
    dinv = lax.rsqrt(deg)
    h2p_ref[...] = jnp.concatenate(
        [h2 * dinv, jnp.zeros((128, D_HID - D_OUT), jnp.float32)], axis=1)
    dinv_ref[...] = dinv


def _tc2(out1p, h, a_s, a_d, denp, degp, b1, W2):
    grid = (N_PAD // 128,)
    return pl.pallas_call(
        _tc2_body,
        grid=grid,
        in_specs=[
            pl.BlockSpec((1, 128, D_HID), lambda i: (0, i, 0)),
            pl.BlockSpec((1, 128, D_HID), lambda i: (1, i, 0)),
            pl.BlockSpec((128, D_HID), lambda i: (i, 0)),
            pl.BlockSpec((128, 1), lambda i: (i, 0)),
            pl.BlockSpec((128, 1), lambda i: (i, 0)),
            pl.BlockSpec((1, 128, 1), lambda i: (0, i, 0)),
            pl.BlockSpec((1, 128, 1), lambda i: (1, i, 0)),
            pl.BlockSpec((1, 128, 1), lambda i: (0, i, 0)),
            pl.BlockSpec((1, 128, 1), lambda i: (1, i, 0)),
            pl.BlockSpec((1, D_HID), lambda i: (0, 0)),
            pl.BlockSpec((D_HID, D_OUT), lambda i: (0, 0)),
        ],
        out_specs=[
            pl.BlockSpec((128, D_HID), lambda i: (i, 0)),
            pl.BlockSpec((128, 1), lambda i: (i, 0)),
        ],
        out_shape=[
            jax.ShapeDtypeStruct((N_PAD, D_HID), jnp.float32),
            jax.ShapeDtypeStruct((N_PAD, 1), jnp.float32),
        ],
    )(out1p.reshape(NC, N_PAD, D_HID), out1p.reshape(NC, N_PAD, D_HID),
      h, a_s, a_d,
      denp.reshape(NC, N_PAD, 1), denp.reshape(NC, N_PAD, 1),
      degp.reshape(NC, N_PAD, 1), degp.reshape(NC, N_PAD, 1),
      b1.reshape(1, D_HID), W2)


def _sc_gcn_body(src2d, dst2d, h2p_hbm, out2_hbm,
                 sidx_v, didx_v, rb0, rb1, acc2, sem0, sem1, ss0, ss1):
    c = lax.axis_index("c")
    s = lax.axis_index("s")

    def zrow(i, _):
        for f in range(D_HID // 16):
            rb0[i, pl.ds(f * 16, 16)] = jnp.zeros((16,), jnp.float32)
        return 0
    lax.fori_loop(0, KC, zrow, 0)
    for t in range(ROWS_PER_TILE // KC):
        pltpu.sync_copy(rb0, acc2.at[pl.ds(s * ROWS_PER_TILE + t * KC, KC)])
    plsc.subcore_barrier()

    chunk0 = jnp.where(c == 0, s * CH0_C, NS * CH0_C + s * CH1_C)
    nblk = jnp.where(c == 0, CH0_C // IDX_BLK, CH1_C // IDX_BLK)

    def blk_body(b, _):
        base = chunk0 + b * IDX_BLK
        pltpu.sync_copy(src2d.at[pl.ds(base, IDX_BLK)], sidx_v)
        pltpu.sync_copy(dst2d.at[pl.ds(base, IDX_BLK)], didx_v)

        pltpu.async_copy(h2p_hbm.at[sidx_v.at[0]], rb0, sem0)

        def pair_body(p, _):
            g0 = 2 * p
            g1 = 2 * p + 1

            @pl.when(p > 0)
            def _():
                pltpu.make_async_copy(rb1, acc2.at[didx_v.at[g1]], ss1).wait()

            pltpu.async_copy(h2p_hbm.at[sidx_v.at[g1]], rb1, sem1)
            pltpu.make_async_copy(h2p_hbm.at[sidx_v.at[g0]], rb0, sem0).wait()
            pltpu.async_copy(rb0, acc2.at[didx_v.at[g0]], ss0, add=True)

            @pl.when(p < IDX_BLK // 2 - 1)
            def _():
                pltpu.make_async_copy(rb0, acc2.at[didx_v.at[g0]], ss0).wait()
                pltpu.async_copy(h2p_hbm.at[sidx_v.at[g0 + 2]], rb0, sem0)

            pltpu.make_async_copy(h2p_hbm.at[sidx_v.at[g1]], rb1, sem1).wait()
            pltpu.async_copy(rb1, acc2.at[didx_v.at[g1]], ss1, add=True)
            return 0

        lax.fori_loop(0, IDX_BLK // 2, pair_body, 0)
        pltpu.make_async_copy(rb0, acc2.at[didx_v.at[0]], ss0).wait()
        pltpu.make_async_copy(rb1, acc2.at[didx_v.at[0]], ss1).wait()
        return 0

    lax.fori_loop(0, nblk, blk_body, 0)
    plsc.subcore_barrier()

    off = c * N_PAD + s * ROWS_PER_TILE
    pltpu.sync_copy(acc2.at[pl.ds(s * ROWS_PER_TILE, ROWS_PER_TILE)],
                    out2_hbm.at[pl.ds(off, ROWS_PER_TILE)])


def _sc_gcn(src2d, dst2d, h2p):
    mesh = plsc.VectorSubcoreMesh(core_axis_name="c", subcore_axis_name="s",
                                  num_cores=NC, num_subcores=NS)
    f = pl.kernel(
        _sc_gcn_body,
        out_type=jax.ShapeDtypeStruct((NC * N_PAD, D_HID), jnp.float32),
        mesh=mesh,
        scratch_types=[
            pltpu.VMEM((IDX_BLK, KC), jnp.int32),
            pltpu.VMEM((IDX_BLK, KC), jnp.int32),
            pltpu.VMEM((KC, D_HID), jnp.float32),
            pltpu.VMEM((KC, D_HID), jnp.float32),
            pltpu.VMEM_SHARED((N_PAD, D_HID), jnp.float32),
            pltpu.SemaphoreType.DMA,
            pltpu.SemaphoreType.DMA,
            pltpu.SemaphoreType.DMA,
            pltpu.SemaphoreType.DMA,
        ],
        compiler_params=pltpu.CompilerParams(needs_layout_passes=False),
    )
    return f(src2d, dst2d, h2p)


def _tc3_body(q0_ref, q1_ref, h2p_ref, dinv_ref, b2_ref, out_ref):
    acc = (q0_ref[0] + q1_ref[0] + h2p_ref[...])[:, :D_OUT]
    out_ref[...] = dinv_ref[...] * acc + b2_ref[...]


def _tc3(out2p, h2p, dinv, b2):
    grid = (N_PAD // 128,)
    return pl.pallas_call(
        _tc3_body,
        grid=grid,
        in_specs=[
            pl.BlockSpec((1, 128, D_HID), lambda i: (0, i, 0)),
            pl.BlockSpec((1, 128, D_HID), lambda i: (1, i, 0)),
            pl.BlockSpec((128, D_HID), lambda i: (i, 0)),
            pl.BlockSpec((128, 1), lambda i: (i, 0)),
            pl.BlockSpec((1, D_OUT), lambda i: (0, 0)),
        ],
        out_specs=pl.BlockSpec((128, D_OUT), lambda i: (i, 0)),
        out_shape=jax.ShapeDtypeStruct((N_PAD, D_OUT), jnp.float32),
    )(out2p.reshape(NC, N_PAD, D_HID), out2p.reshape(NC, N_PAD, D_HID),
      h2p, dinv, b2.reshape(1, D_OUT))


def kernel(x, edge_index, W1, att_src, att_dst, b1, W2, b2):
    x_pad = jnp.pad(x, ((0, N_PAD - N_NODES), (0, 0)))
    src = edge_index[0].astype(jnp.int32)
    dst = edge_index[1].astype(jnp.int32)
    pad = jnp.full((E_PAD - N_EDGES,), N_PAD - 1, jnp.int32)
    src_flat = jnp.concatenate([src, pad])
    dst_flat = jnp.concatenate([dst, pad])
    src_g = src_flat.reshape(NW * CHUNKS_G, KG)
    dst_g = dst_flat.reshape(NW * CHUNKS_G, KG)
    src_c = src_flat.reshape(NW * CHUNKS_C, KC)
    dst_c = dst_flat.reshape(NW * CHUNKS_C, KC)

    h, a_s, a_d = _tc1(x_pad, W1, att_src, att_dst)
    out1p, denp, degp = _sc_gat(src_g, dst_g,
                                a_s.reshape(N_PAD), a_d.reshape(N_PAD), h)
    h2p, dinv = _tc2(out1p, h, a_s, a_d, denp, degp, b1, W2)
    out2p = _sc_gcn(src_c, dst_c, h2p)
    out2 = _tc3(out2p, h2p, dinv, b2)
    return out2[:N_NODES]
